# Initial kernel scaffold; baseline (speedup 1.0000x reference)
import functools

import jax
import jax.numpy as jnp
from jax import lax
from jax.experimental import pallas as pl
from jax.experimental.pallas import tpu as pltpu
from jax.experimental.pallas import tpu_sc as plsc

N_NODES = 50000
D = 512
B = 1024
S1 = 25
S2 = 10

NG = B + B * S2
NW = 32
PER_W = NG // NW
GCHUNK = 88
NGC = PER_W // GCHUNK
SGRP = 4
NSC = PER_W // SGRP
SROWS = SGRP * S1


def _sc_gather_sum(x, gidx, sidx):
  mesh = plsc.VectorSubcoreMesh(core_axis_name="c", subcore_axis_name="s")

  @functools.partial(
      pl.kernel,
      out_type=(
          jax.ShapeDtypeStruct((NG, D), jnp.float32),
          jax.ShapeDtypeStruct((NG, D), jnp.float32),
      ),
      mesh=mesh,
      scratch_types=[
          pltpu.VMEM((GCHUNK,), jnp.int32),
          pltpu.VMEM((GCHUNK, D), jnp.float32),
          pltpu.VMEM((SROWS,), jnp.int32),
          pltpu.VMEM((SROWS, D), jnp.float32),
          pltpu.VMEM((SGRP, D), jnp.float32),
          pltpu.SemaphoreType.DMA,
      ],
  )
  def k(x_hbm, gidx_hbm, sidx_hbm, gout_hbm, sout_hbm,
        gi_v, grows_v, si_v, srows_v, sums_v, sem):
    wid = lax.axis_index("s") * 2 + lax.axis_index("c")
    wbase = wid * PER_W

    def gather_chunk(i, _):
      base = wbase + i * GCHUNK
      pltpu.sync_copy(gidx_hbm.at[pl.ds(base, GCHUNK)], gi_v)
      pltpu.async_copy(x_hbm.at[gi_v], grows_v, sem).wait()
      pltpu.sync_copy(grows_v, gout_hbm.at[pl.ds(base, GCHUNK)])
      return 0

    lax.fori_loop(0, NGC, gather_chunk, 0)

    def sum_chunk(i, _):
      base = wbase + i * SGRP
      pltpu.sync_copy(sidx_hbm.at[pl.ds(base * S1, SROWS)], si_v)
      pltpu.async_copy(x_hbm.at[si_v], srows_v, sem).wait()
      for g in range(SGRP):
        def col_chunk(c, _):
          acc = jnp.zeros((16,), jnp.float32)
          for r in range(S1):
            acc = acc + srows_v[g * S1 + r, pl.ds(c * 16, 16)]
          sums_v[g, pl.ds(c * 16, 16)] = acc
          return 0
        lax.fori_loop(0, D // 16, col_chunk, 0)
      pltpu.sync_copy(sums_v, sout_hbm.at[pl.ds(base, SGRP)])
      return 0

    lax.fori_loop(0, NSC, sum_chunk, 0)

  return k(x, gidx, sidx)


def _l2norm(h):
  n2 = jnp.sum(h * h, axis=-1, keepdims=True)
  return h * jnp.where(n2 > 0, lax.rsqrt(n2), 1.0)


def _tc_layers(h0_t, sum_t, h0_n, sum_n, W0a, W0b, W1a, W1b, b0, b1):
  BLK = 128
  NBLK = B // BLK

  def body(h0t_r, st_r, h0n_r, sn_r, w0a_r, w0b_r, w1a_r, w1b_r,
           b0_r, b1_r, z_r):
    inv_s1 = 1.0 / S1
    w0a = w0a_r[...]
    w0b = w0b_r[...]
    h1t = h0t_r[...] @ w0a + (st_r[...] * inv_s1) @ w0b + b0_r[...]
    h1t = _l2norm(jnp.maximum(h1t, 0.0))
    h1n = h0n_r[...] @ w0a + (sn_r[...] * inv_s1) @ w0b + b0_r[...]
    h1n = _l2norm(jnp.maximum(h1n, 0.0))
    agg2 = jnp.mean(h1n.reshape(BLK, S2, D), axis=1)
    z = h1t @ w1a_r[...] + agg2 @ w1b_r[...] + b1_r[...]
    z_r[...] = _l2norm(jnp.maximum(z, 0.0))

  full = lambda i: (0, 0)
  return pl.pallas_call(
      body,
      grid=(NBLK,),
      in_specs=[
          pl.BlockSpec((BLK, D), lambda i: (i, 0)),
          pl.BlockSpec((BLK, D), lambda i: (i, 0)),
          pl.BlockSpec((BLK * S2, D), lambda i: (i, 0)),
          pl.BlockSpec((BLK * S2, D), lambda i: (i, 0)),
          pl.BlockSpec((D, D), full),
          pl.BlockSpec((D, D), full),
          pl.BlockSpec((D, D), full),
          pl.BlockSpec((D, D), full),
          pl.BlockSpec((1, D), full),
          pl.BlockSpec((1, D), full),
      ],
      out_specs=pl.BlockSpec((BLK, D), lambda i: (i, 0)),
      out_shape=jax.ShapeDtypeStruct((B, D), jnp.float32),
  )(h0_t, sum_t, h0_n, sum_n, W0a, W0b, W1a, W1b, b0, b1)


def kernel(x, nodes, nb1, nb0_t, nb0_n, W0, b0, W1, b1):
  gidx = jnp.concatenate([nodes, nb1.reshape(-1)]).astype(jnp.int32)
  sidx = jnp.concatenate(
      [nb0_t.reshape(-1), nb0_n.reshape(-1)]).astype(jnp.int32)

  gout, sout = _sc_gather_sum(x, gidx, sidx)

  h0_t, h0_n = gout[:B], gout[B:]
  sum_t, sum_n = sout[:B], sout[B:]

  z = _tc_layers(
      h0_t, sum_t, h0_n, sum_n,
      W0[:D], W0[D:], W1[:D], W1[D:],
      b0.reshape(1, D), b1.reshape(1, D),
  )
  return z


# speedup vs baseline: 2.4987x; 2.4987x over previous
import functools

import jax
import jax.numpy as jnp
from jax import lax
from jax.experimental import pallas as pl
from jax.experimental.pallas import tpu as pltpu
from jax.experimental.pallas import tpu_sc as plsc

N_NODES = 50000
D = 512
B = 1024
S1 = 25
S2 = 10

NG = B + B * S2
NW = 32
PER_W = NG // NW
GCHUNK = 16
NGC = PER_W // GCHUNK
SGRP = 8
NSC = PER_W // SGRP
SROWS = SGRP * S1
SHALF = 104


def _sc_gather_sum(x, gidx, sidx):
  mesh = plsc.VectorSubcoreMesh(
      core_axis_name="c", subcore_axis_name="s", num_cores=2, num_subcores=16)

  @functools.partial(
      pl.kernel,
      out_type=(
          jax.ShapeDtypeStruct((NG, D), jnp.float32),
          jax.ShapeDtypeStruct((NG, D), jnp.float32),
      ),
      mesh=mesh,
      scratch_types=[
          pltpu.VMEM((GCHUNK,), jnp.int32),
          pltpu.VMEM((GCHUNK, D), jnp.float32),
          pltpu.VMEM((SROWS,), jnp.int32),
          pltpu.VMEM((SROWS, D), jnp.float32),
          pltpu.VMEM((SGRP, D), jnp.float32),
          pltpu.SemaphoreType.DMA,
      ],
  )
  def k(x_hbm, gidx_hbm, sidx_hbm, gout_hbm, sout_hbm,
        gi_v, grows_v, si_v, srows_v, sums_v, sem):
    wid = lax.axis_index("s") * 2 + lax.axis_index("c")
    wbase = wid * PER_W

    def gather_chunk(i, _):
      base = wbase + i * GCHUNK
      pltpu.sync_copy(gidx_hbm.at[pl.ds(base, GCHUNK)], gi_v)
      pltpu.async_copy(x_hbm.at[gi_v], grows_v, sem).wait()
      pltpu.sync_copy(grows_v, gout_hbm.at[pl.ds(base, GCHUNK)])
      return 0

    lax.fori_loop(0, NGC, gather_chunk, 0)

    def sum_chunk(i, _):
      base = wbase + i * SGRP
      pltpu.sync_copy(sidx_hbm.at[pl.ds(base * S1, SROWS)], si_v)
      c0 = pltpu.async_copy(
          x_hbm.at[si_v.at[pl.ds(0, SHALF)]],
          srows_v.at[pl.ds(0, SHALF)], sem)
      c1 = pltpu.async_copy(
          x_hbm.at[si_v.at[pl.ds(SHALF, SROWS - SHALF)]],
          srows_v.at[pl.ds(SHALF, SROWS - SHALF)], sem)
      c0.wait()
      c1.wait()
      for g in range(SGRP):
        def col_chunk(c, _):
          acc = jnp.zeros((16,), jnp.float32)
          for r in range(S1):
            acc = acc + srows_v[g * S1 + r, pl.ds(c * 16, 16)]
          sums_v[g, pl.ds(c * 16, 16)] = acc
          return 0
        lax.fori_loop(0, D // 16, col_chunk, 0)
      pltpu.sync_copy(sums_v, sout_hbm.at[pl.ds(base, SGRP)])
      return 0

    lax.fori_loop(0, NSC, sum_chunk, 0)

  return k(x, gidx, sidx)


def _l2norm(h):
  n2 = jnp.sum(h * h, axis=-1, keepdims=True)
  return h * jnp.where(n2 > 0, lax.rsqrt(n2), 1.0)


def _tc_layers(h0_t, sum_t, h0_n, sum_n, W0a, W0b, W1a, W1b, b0, b1):
  BLK = 128
  NBLK = B // BLK

  def body(h0t_r, st_r, h0n_r, sn_r, w0a_r, w0b_r, w1a_r, w1b_r,
           b0_r, b1_r, z_r):
    inv_s1 = 1.0 / S1
    w0a = w0a_r[...]
    w0b = w0b_r[...]
    h1t = h0t_r[...] @ w0a + (st_r[...] * inv_s1) @ w0b + b0_r[...]
    h1t = _l2norm(jnp.maximum(h1t, 0.0))
    h1n = h0n_r[...] @ w0a + (sn_r[...] * inv_s1) @ w0b + b0_r[...]
    h1n = _l2norm(jnp.maximum(h1n, 0.0))
    agg2 = jnp.mean(h1n.reshape(BLK, S2, D), axis=1)
    z = h1t @ w1a_r[...] + agg2 @ w1b_r[...] + b1_r[...]
    z_r[...] = _l2norm(jnp.maximum(z, 0.0))

  full = lambda i: (0, 0)
  return pl.pallas_call(
      body,
      grid=(NBLK,),
      in_specs=[
          pl.BlockSpec((BLK, D), lambda i: (i, 0)),
          pl.BlockSpec((BLK, D), lambda i: (i, 0)),
          pl.BlockSpec((BLK * S2, D), lambda i: (i, 0)),
          pl.BlockSpec((BLK * S2, D), lambda i: (i, 0)),
          pl.BlockSpec((D, D), full),
          pl.BlockSpec((D, D), full),
          pl.BlockSpec((D, D), full),
          pl.BlockSpec((D, D), full),
          pl.BlockSpec((1, D), full),
          pl.BlockSpec((1, D), full),
      ],
      out_specs=pl.BlockSpec((BLK, D), lambda i: (i, 0)),
      out_shape=jax.ShapeDtypeStruct((B, D), jnp.float32),
  )(h0_t, sum_t, h0_n, sum_n, W0a, W0b, W1a, W1b, b0, b1)


def kernel(x, nodes, nb1, nb0_t, nb0_n, W0, b0, W1, b1):
  gidx = jnp.concatenate([nodes, nb1.reshape(-1)]).astype(jnp.int32)
  sidx = jnp.concatenate(
      [nb0_t.reshape(-1), nb0_n.reshape(-1)]).astype(jnp.int32)

  gout, sout = _sc_gather_sum(x, gidx, sidx)

  h0_t, h0_n = gout[:B], gout[B:]
  sum_t, sum_n = sout[:B], sout[B:]

  z = _tc_layers(
      h0_t, sum_t, h0_n, sum_n,
      W0[:D], W0[D:], W1[:D], W1[D:],
      b0.reshape(1, D), b1.reshape(1, D),
  )
  return z
